# baseline (speedup 1.0000x reference)
import functools
import jax
import jax.numpy as jnp
from jax import lax
from jax.experimental import pallas as pl
from jax.experimental.pallas import tpu as pltpu
from jax.experimental.pallas import tpu_sc as plsc

N = 10000
E = 320000
D = 128
DE = 16
DEPTH = 3

NC = 2
NS = 16
EPT = E // (NC * NS)
CH = 40
NCHUNK = EPT // CH
G = 80
NG = N // G

NB = 2000
BE = 8000


def _sc_body(hA, srcg, dstg, eB, zeros, out,
             si0, si1, sd0, sd1, g0, g1, e0, e1, agg,
             ssi0, ssi1, ssd0, ssd1, sg0, sg1, se0, se1):
    c = lax.axis_index("c")
    s = lax.axis_index("s")
    wid = c * NS + s
    base0 = wid * EPT

    sibufs = (si0, si1)
    sdbufs = (sd0, sd1)
    gbufs = (g0, g1)
    ebufs = (e0, e1)
    sisems = (ssi0, ssi1)
    sdsems = (ssd0, ssd1)
    gsems = (sg0, sg1)
    esems = (se0, se1)

    def issue_src(k, b):
        pltpu.async_copy(srcg.at[pl.ds(base0 + k * CH, CH)],
                         sibufs[b], sisems[b])

    def issue_dst(k, b):
        pltpu.async_copy(dstg.at[pl.ds(base0 + k * CH, CH)],
                         sdbufs[b], sdsems[b])

    def issue_main(k, b):
        pltpu.async_copy(hA.at[sibufs[b]], gbufs[b], gsems[b])
        pltpu.async_copy(eB.at[pl.ds(base0 + k * CH, CH)], ebufs[b], esems[b])

    def wait_main(k, b):
        pltpu.make_async_copy(hA.at[sibufs[b]], gbufs[b], gsems[b]).wait()
        pltpu.make_async_copy(eB.at[pl.ds(base0 + k * CH, CH)],
                              ebufs[b], esems[b]).wait()

    issue_src(0, 0)
    issue_src(1, 1)
    issue_dst(0, 0)
    pltpu.make_async_copy(srcg.at[pl.ds(base0, CH)], si0, ssi0).wait()
    issue_main(0, 0)

    def init_body(j, carry):
        k = s + NS * j

        @pl.when(k < NG)
        def _():
            pltpu.sync_copy(zeros.at[pl.ds(k * G, G)], agg.at[pl.ds(k * G, G)])

        return carry

    lax.fori_loop(0, (NG + NS - 1) // NS, init_body, 0)
    plsc.subcore_barrier()

    def process(k, b):
        nb = 1 - b

        @pl.when(k + 1 < NCHUNK)
        def _():
            pltpu.make_async_copy(srcg.at[pl.ds(base0, CH)],
                                  sibufs[nb], sisems[nb]).wait()
            issue_main(k + 1, nb)
            issue_dst(k + 1, nb)

        wait_main(k, b)

        @pl.when(k + 2 < NCHUNK)
        def _():
            issue_src(k + 2, b)

        gath_v = gbufs[b]
        eB_v = ebufs[b]

        def row_body(r, rcarry):
            for cc in range(D // 16):
                z = gath_v[r, pl.ds(cc * 16, 16)] + eB_v[r, pl.ds(cc * 16, 16)]
                gath_v[r, pl.ds(cc * 16, 16)] = jnp.maximum(z, 0.1 * z)
            return rcarry

        lax.fori_loop(0, CH, row_body, 0)

        pltpu.make_async_copy(dstg.at[pl.ds(base0, CH)],
                              sdbufs[b], sdsems[b]).wait()
        pltpu.sync_copy(gath_v, agg.at[sdbufs[b]], add=True)

    def chunk_body(i, carry):
        process(2 * i, 0)
        process(2 * i + 1, 1)
        return carry

    lax.fori_loop(0, NCHUNK // 2, chunk_body, 0)
    plsc.subcore_barrier()

    def wb_body(j, carry):
        k = s + NS * j

        @pl.when(k < NG)
        def _():
            pltpu.sync_copy(agg.at[pl.ds(k * G, G)],
                            out.at[c, pl.ds(k * G, G)])

        return carry

    lax.fori_loop(0, (NG + NS - 1) // NS, wb_body, 0)


_sc_aggregate = pl.kernel(
    _sc_body,
    out_type=jax.ShapeDtypeStruct((NC, N, D), jnp.float32),
    mesh=plsc.VectorSubcoreMesh(core_axis_name="c", subcore_axis_name="s",
                                num_cores=NC, num_subcores=NS),
    scratch_types=(
        [pltpu.VMEM((CH,), jnp.int32)] * 4
        + [pltpu.VMEM((CH, D), jnp.float32)] * 4
        + [pltpu.VMEM_SHARED((N, D), jnp.float32)]
        + [pltpu.SemaphoreType.DMA] * 8
    ),
)


def _prep_body(x_r, wp_r, bp_r, u2a_r, h_r, ha_r):
    h = jnp.dot(x_r[...], wp_r[...], preferred_element_type=jnp.float32)
    h = h + bp_r[...]
    h_r[...] = h
    ha_r[...] = jnp.dot(h, u2a_r[...], preferred_element_type=jnp.float32)


_tc_prep = pl.pallas_call(
    _prep_body,
    grid=(N // NB,),
    in_specs=[
        pl.BlockSpec((NB, D), lambda i: (i, 0)),
        pl.BlockSpec((D, D), lambda i: (0, 0)),
        pl.BlockSpec((1, D), lambda i: (0, 0)),
        pl.BlockSpec((D, D), lambda i: (0, 0)),
    ],
    out_specs=[
        pl.BlockSpec((NB, D), lambda i: (i, 0)),
        pl.BlockSpec((NB, D), lambda i: (i, 0)),
    ],
    out_shape=[
        jax.ShapeDtypeStruct((N, D), jnp.float32),
        jax.ShapeDtypeStruct((N, D), jnp.float32),
    ],
)


def _eb_body(ea_r, u2e_r, b2_r, eb_r):
    eb_r[...] = (jnp.dot(ea_r[...], u2e_r[...],
                         preferred_element_type=jnp.float32) + b2_r[...])


_tc_eb = pl.pallas_call(
    _eb_body,
    grid=(E // BE,),
    in_specs=[
        pl.BlockSpec((BE, DE), lambda j: (j, 0)),
        pl.BlockSpec((DE, D), lambda j: (0, 0)),
        pl.BlockSpec((1, D), lambda j: (0, 0)),
    ],
    out_specs=pl.BlockSpec((BE, D), lambda j: (j, 0)),
    out_shape=jax.ShapeDtypeStruct((E, D), jnp.float32),
)


def _upd_body_mid(h_r, agg_r, u1a_r, u1b_r, b1_r, u2a_r, hn_r, han_r):
    a = agg_r[0] + agg_r[1]
    hn = jnp.dot(h_r[...], u1a_r[...], preferred_element_type=jnp.float32)
    hn = hn + jnp.dot(a, u1b_r[...], preferred_element_type=jnp.float32)
    hn = hn + b1_r[...]
    hn_r[...] = hn
    han_r[...] = jnp.dot(hn, u2a_r[...], preferred_element_type=jnp.float32)


_tc_update_mid = pl.pallas_call(
    _upd_body_mid,
    grid=(N // NB,),
    in_specs=[
        pl.BlockSpec((NB, D), lambda i: (i, 0)),
        pl.BlockSpec((NC, NB, D), lambda i: (0, i, 0)),
        pl.BlockSpec((D, D), lambda i: (0, 0)),
        pl.BlockSpec((D, D), lambda i: (0, 0)),
        pl.BlockSpec((1, D), lambda i: (0, 0)),
        pl.BlockSpec((D, D), lambda i: (0, 0)),
    ],
    out_specs=[
        pl.BlockSpec((NB, D), lambda i: (i, 0)),
        pl.BlockSpec((NB, D), lambda i: (i, 0)),
    ],
    out_shape=[
        jax.ShapeDtypeStruct((N, D), jnp.float32),
        jax.ShapeDtypeStruct((N, D), jnp.float32),
    ],
)


def _upd_body_last(h_r, agg_r, u1a_r, u1b_r, b1_r, hn_r):
    a = agg_r[0] + agg_r[1]
    hn = jnp.dot(h_r[...], u1a_r[...], preferred_element_type=jnp.float32)
    hn = hn + jnp.dot(a, u1b_r[...], preferred_element_type=jnp.float32)
    hn_r[...] = hn + b1_r[...]


_tc_update_last = pl.pallas_call(
    _upd_body_last,
    grid=(N // NB,),
    in_specs=[
        pl.BlockSpec((NB, D), lambda i: (i, 0)),
        pl.BlockSpec((NC, NB, D), lambda i: (0, i, 0)),
        pl.BlockSpec((D, D), lambda i: (0, 0)),
        pl.BlockSpec((D, D), lambda i: (0, 0)),
        pl.BlockSpec((1, D), lambda i: (0, 0)),
    ],
    out_specs=pl.BlockSpec((NB, D), lambda i: (i, 0)),
    out_shape=jax.ShapeDtypeStruct((N, D), jnp.float32),
)


def kernel(x, edge_index, edge_attr, W_proj, b_proj, U2_W, U2_b, U1_W, U1_b):
    src = edge_index[0]
    dst = edge_index[1]
    bp = b_proj.reshape(1, D)
    zeros = jnp.zeros((N, D), jnp.float32)

    h, hA = _tc_prep(x, W_proj, bp, U2_W[0, :D, :])
    eBs = [_tc_eb(edge_attr, U2_W[i, D:, :], U2_b[i].reshape(1, D))
           for i in range(DEPTH)]
    for i in range(DEPTH):
        agg = _sc_aggregate(hA, src, dst, eBs[i], zeros)
        if i < DEPTH - 1:
            h, hA = _tc_update_mid(h, agg, U1_W[i, :D, :], U1_W[i, D:, :],
                                   U1_b[i].reshape(1, D), U2_W[i + 1, :D, :])
        else:
            h = _tc_update_last(h, agg, U1_W[i, :D, :], U1_W[i, D:, :],
                                U1_b[i].reshape(1, D))
    return h


# speedup vs baseline: 1.0716x; 1.0716x over previous
import functools
import jax
import jax.numpy as jnp
from jax import lax
from jax.experimental import pallas as pl
from jax.experimental.pallas import tpu as pltpu
from jax.experimental.pallas import tpu_sc as plsc

N = 10000
E = 320000
D = 128
DE = 16
DEPTH = 3

NC = 2
NS = 16
EPT = E // (NC * NS)
CH = 40
NCHUNK = EPT // CH
G = 80
NG = N // G

NB = 2000
BE = 8000


def _sc_body(hA, srcg, dstg, eB, zeros, out,
             si0, si1, si2, sd0, sd1, sd2, g0, g1, g2, e0, e1, e2, agg,
             ssi0, ssi1, ssi2, ssd0, ssd1, ssd2,
             sg0, sg1, sg2, se0, se1, se2, sct0, sct1, sct2):
    c = lax.axis_index("c")
    s = lax.axis_index("s")
    wid = c * NS + s
    base0 = wid * EPT

    sibufs = (si0, si1, si2)
    sdbufs = (sd0, sd1, sd2)
    gbufs = (g0, g1, g2)
    ebufs = (e0, e1, e2)
    sisems = (ssi0, ssi1, ssi2)
    sdsems = (ssd0, ssd1, ssd2)
    gsems = (sg0, sg1, sg2)
    esems = (se0, se1, se2)
    sctsems = (sct0, sct1, sct2)

    def issue_src(k, b):
        pltpu.async_copy(srcg.at[pl.ds(base0 + k * CH, CH)],
                         sibufs[b], sisems[b])

    def issue_dst(k, b):
        pltpu.async_copy(dstg.at[pl.ds(base0 + k * CH, CH)],
                         sdbufs[b], sdsems[b])

    def issue_main(k, b):
        pltpu.async_copy(hA.at[sibufs[b]], gbufs[b], gsems[b])
        pltpu.async_copy(eB.at[pl.ds(base0 + k * CH, CH)], ebufs[b], esems[b])

    def wait_main(k, b):
        pltpu.make_async_copy(hA.at[sibufs[b]], gbufs[b], gsems[b]).wait()
        pltpu.make_async_copy(eB.at[pl.ds(base0 + k * CH, CH)],
                              ebufs[b], esems[b]).wait()

    def wait_src(b):
        pltpu.make_async_copy(srcg.at[pl.ds(base0, CH)],
                              sibufs[b], sisems[b]).wait()

    def wait_sct(b):
        pltpu.make_async_copy(gbufs[b], agg.at[sdbufs[b]], sctsems[b]).wait()

    issue_src(0, 0)
    issue_src(1, 1)
    issue_src(2, 2)
    issue_dst(0, 0)
    issue_dst(1, 1)
    wait_src(0)
    issue_main(0, 0)
    wait_src(1)
    issue_main(1, 1)

    def init_body(j, carry):
        k = s + NS * j

        @pl.when(k < NG)
        def _():
            pltpu.sync_copy(zeros.at[pl.ds(k * G, G)], agg.at[pl.ds(k * G, G)])

        return carry

    lax.fori_loop(0, (NG + NS - 1) // NS, init_body, 0)
    plsc.subcore_barrier()

    def process(k, b):
        nb2 = (b + 2) % 3

        wait_main(k, b)

        @pl.when(k + 3 < NCHUNK)
        def _():
            issue_src(k + 3, b)

        gath_v = gbufs[b]
        eB_v = ebufs[b]

        def row_body(r, rcarry):
            for cc in range(D // 16):
                z = gath_v[r, pl.ds(cc * 16, 16)] + eB_v[r, pl.ds(cc * 16, 16)]
                gath_v[r, pl.ds(cc * 16, 16)] = jnp.maximum(z, 0.1 * z)
            return rcarry

        lax.fori_loop(0, CH, row_body, 0)

        @pl.when((k >= 1) & (k + 2 < NCHUNK))
        def _():
            wait_sct(nb2)

        @pl.when(k + 2 < NCHUNK)
        def _():
            wait_src(nb2)
            issue_main(k + 2, nb2)
            issue_dst(k + 2, nb2)

        pltpu.make_async_copy(dstg.at[pl.ds(base0, CH)],
                              sdbufs[b], sdsems[b]).wait()
        pltpu.async_copy(gath_v, agg.at[sdbufs[b]], sctsems[b], add=True)

    def chunk_body(i, carry):
        process(3 * i, 0)
        process(3 * i + 1, 1)
        process(3 * i + 2, 2)
        return carry

    lax.fori_loop(0, (NCHUNK - 1) // 3, chunk_body, 0)
    process(NCHUNK - 1, (NCHUNK - 1) % 3)
    wait_sct((NCHUNK - 3) % 3)
    wait_sct((NCHUNK - 2) % 3)
    wait_sct((NCHUNK - 1) % 3)
    plsc.subcore_barrier()

    def wb_body(j, carry):
        k = s + NS * j

        @pl.when(k < NG)
        def _():
            pltpu.sync_copy(agg.at[pl.ds(k * G, G)],
                            out.at[c, pl.ds(k * G, G)])

        return carry

    lax.fori_loop(0, (NG + NS - 1) // NS, wb_body, 0)


_sc_aggregate = pl.kernel(
    _sc_body,
    out_type=jax.ShapeDtypeStruct((NC, N, D), jnp.float32),
    mesh=plsc.VectorSubcoreMesh(core_axis_name="c", subcore_axis_name="s",
                                num_cores=NC, num_subcores=NS),
    scratch_types=(
        [pltpu.VMEM((CH,), jnp.int32)] * 6
        + [pltpu.VMEM((CH, D), jnp.float32)] * 6
        + [pltpu.VMEM_SHARED((N, D), jnp.float32)]
        + [pltpu.SemaphoreType.DMA] * 15
    ),
)


def _prep_body(x_r, wp_r, bp_r, u2a_r, h_r, ha_r):
    h = jnp.dot(x_r[...], wp_r[...], preferred_element_type=jnp.float32)
    h = h + bp_r[...]
    h_r[...] = h
    ha_r[...] = jnp.dot(h, u2a_r[...], preferred_element_type=jnp.float32)


_tc_prep = pl.pallas_call(
    _prep_body,
    grid=(N // NB,),
    in_specs=[
        pl.BlockSpec((NB, D), lambda i: (i, 0)),
        pl.BlockSpec((D, D), lambda i: (0, 0)),
        pl.BlockSpec((1, D), lambda i: (0, 0)),
        pl.BlockSpec((D, D), lambda i: (0, 0)),
    ],
    out_specs=[
        pl.BlockSpec((NB, D), lambda i: (i, 0)),
        pl.BlockSpec((NB, D), lambda i: (i, 0)),
    ],
    out_shape=[
        jax.ShapeDtypeStruct((N, D), jnp.float32),
        jax.ShapeDtypeStruct((N, D), jnp.float32),
    ],
)


def _eb_body(ea_r, u2e_r, b2_r, eb_r):
    eb_r[...] = (jnp.dot(ea_r[...], u2e_r[...],
                         preferred_element_type=jnp.float32) + b2_r[...])


_tc_eb = pl.pallas_call(
    _eb_body,
    grid=(E // BE,),
    in_specs=[
        pl.BlockSpec((BE, DE), lambda j: (j, 0)),
        pl.BlockSpec((DE, D), lambda j: (0, 0)),
        pl.BlockSpec((1, D), lambda j: (0, 0)),
    ],
    out_specs=pl.BlockSpec((BE, D), lambda j: (j, 0)),
    out_shape=jax.ShapeDtypeStruct((E, D), jnp.float32),
)


def _upd_body_mid(h_r, agg_r, u1a_r, u1b_r, b1_r, u2a_r, hn_r, han_r):
    a = agg_r[0] + agg_r[1]
    hn = jnp.dot(h_r[...], u1a_r[...], preferred_element_type=jnp.float32)
    hn = hn + jnp.dot(a, u1b_r[...], preferred_element_type=jnp.float32)
    hn = hn + b1_r[...]
    hn_r[...] = hn
    han_r[...] = jnp.dot(hn, u2a_r[...], preferred_element_type=jnp.float32)


_tc_update_mid = pl.pallas_call(
    _upd_body_mid,
    grid=(N // NB,),
    in_specs=[
        pl.BlockSpec((NB, D), lambda i: (i, 0)),
        pl.BlockSpec((NC, NB, D), lambda i: (0, i, 0)),
        pl.BlockSpec((D, D), lambda i: (0, 0)),
        pl.BlockSpec((D, D), lambda i: (0, 0)),
        pl.BlockSpec((1, D), lambda i: (0, 0)),
        pl.BlockSpec((D, D), lambda i: (0, 0)),
    ],
    out_specs=[
        pl.BlockSpec((NB, D), lambda i: (i, 0)),
        pl.BlockSpec((NB, D), lambda i: (i, 0)),
    ],
    out_shape=[
        jax.ShapeDtypeStruct((N, D), jnp.float32),
        jax.ShapeDtypeStruct((N, D), jnp.float32),
    ],
)


def _upd_body_last(h_r, agg_r, u1a_r, u1b_r, b1_r, hn_r):
    a = agg_r[0] + agg_r[1]
    hn = jnp.dot(h_r[...], u1a_r[...], preferred_element_type=jnp.float32)
    hn = hn + jnp.dot(a, u1b_r[...], preferred_element_type=jnp.float32)
    hn_r[...] = hn + b1_r[...]


_tc_update_last = pl.pallas_call(
    _upd_body_last,
    grid=(N // NB,),
    in_specs=[
        pl.BlockSpec((NB, D), lambda i: (i, 0)),
        pl.BlockSpec((NC, NB, D), lambda i: (0, i, 0)),
        pl.BlockSpec((D, D), lambda i: (0, 0)),
        pl.BlockSpec((D, D), lambda i: (0, 0)),
        pl.BlockSpec((1, D), lambda i: (0, 0)),
    ],
    out_specs=pl.BlockSpec((NB, D), lambda i: (i, 0)),
    out_shape=jax.ShapeDtypeStruct((N, D), jnp.float32),
)


def kernel(x, edge_index, edge_attr, W_proj, b_proj, U2_W, U2_b, U1_W, U1_b):
    src = edge_index[0]
    dst = edge_index[1]
    bp = b_proj.reshape(1, D)
    zeros = jnp.zeros((N, D), jnp.float32)

    h, hA = _tc_prep(x, W_proj, bp, U2_W[0, :D, :])
    eBs = [_tc_eb(edge_attr, U2_W[i, D:, :], U2_b[i].reshape(1, D))
           for i in range(DEPTH)]
    for i in range(DEPTH):
        agg = _sc_aggregate(hA, src, dst, eBs[i], zeros)
        if i < DEPTH - 1:
            h, hA = _tc_update_mid(h, agg, U1_W[i, :D, :], U1_W[i, D:, :],
                                   U1_b[i].reshape(1, D), U2_W[i + 1, :D, :])
        else:
            h = _tc_update_last(h, agg, U1_W[i, :D, :], U1_W[i, D:, :],
                                U1_b[i].reshape(1, D))
    return h


# speedup vs baseline: 1.1323x; 1.0566x over previous
import functools
import jax
import jax.numpy as jnp
from jax import lax
from jax.experimental import pallas as pl
from jax.experimental.pallas import tpu as pltpu
from jax.experimental.pallas import tpu_sc as plsc

N = 10000
E = 320000
D = 128
DE = 16
DEPTH = 3

NC = 2
NS = 16
EPT = E // (NC * NS)
CH = 40
NCHUNK = EPT // CH
G = 80
NG = N // G

NB = 2000
BE = 8000


def _sc_body(hA, srcg, dstg, eB, zeros, out,
             si0, si1, si2, sd0, sd1, sd2, g0, g1, g2, e0, e1, e2, agg,
             ssi0, ssi1, ssi2, ssd0, ssd1, ssd2,
             sg0, sg1, sg2, se0, se1, se2, sct0, sct1, sct2):
    c = lax.axis_index("c")
    s = lax.axis_index("s")
    wid = c * NS + s
    base0 = wid * EPT

    sibufs = (si0, si1, si2)
    sdbufs = (sd0, sd1, sd2)
    gbufs = (g0, g1, g2)
    ebufs = (e0, e1, e2)
    sisems = (ssi0, ssi1, ssi2)
    sdsems = (ssd0, ssd1, ssd2)
    gsems = (sg0, sg1, sg2)
    esems = (se0, se1, se2)
    sctsems = (sct0, sct1, sct2)

    def issue_src(k, b):
        pltpu.async_copy(srcg.at[pl.ds(base0 + k * CH, CH)],
                         sibufs[b], sisems[b])

    def issue_dst(k, b):
        pltpu.async_copy(dstg.at[pl.ds(base0 + k * CH, CH)],
                         sdbufs[b], sdsems[b])

    def issue_main(k, b):
        pltpu.async_copy(hA.at[sibufs[b]], gbufs[b], gsems[b])
        pltpu.async_copy(eB.at[pl.ds(base0 + k * CH, CH)], ebufs[b], esems[b])

    def wait_main(k, b):
        pltpu.make_async_copy(hA.at[sibufs[b]], gbufs[b], gsems[b]).wait()
        pltpu.make_async_copy(eB.at[pl.ds(base0 + k * CH, CH)],
                              ebufs[b], esems[b]).wait()

    def wait_src(b):
        pltpu.make_async_copy(srcg.at[pl.ds(base0, CH)],
                              sibufs[b], sisems[b]).wait()

    def wait_sct(b):
        pltpu.make_async_copy(gbufs[b], agg.at[sdbufs[b]], sctsems[b]).wait()

    issue_src(0, 0)
    issue_src(1, 1)
    issue_src(2, 2)
    issue_dst(0, 0)
    issue_dst(1, 1)
    wait_src(0)
    issue_main(0, 0)
    wait_src(1)
    issue_main(1, 1)

    def init_body(j, carry):
        k = s + NS * j

        @pl.when(k < NG)
        def _():
            pltpu.sync_copy(zeros.at[pl.ds(k * G, G)], agg.at[pl.ds(k * G, G)])

        return carry

    lax.fori_loop(0, (NG + NS - 1) // NS, init_body, 0)
    plsc.subcore_barrier()

    def process(k, b):
        nb2 = (b + 2) % 3

        wait_main(k, b)

        @pl.when(k + 3 < NCHUNK)
        def _():
            issue_src(k + 3, b)

        @pl.when((k >= 1) & (k + 2 < NCHUNK))
        def _():
            wait_sct(nb2)

        @pl.when(k + 2 < NCHUNK)
        def _():
            wait_src(nb2)
            issue_main(k + 2, nb2)
            issue_dst(k + 2, nb2)

        gath_v = gbufs[b]
        eB_v = ebufs[b]

        def row_body(r, rcarry):
            for cc in range(D // 16):
                z = gath_v[r, pl.ds(cc * 16, 16)] + eB_v[r, pl.ds(cc * 16, 16)]
                gath_v[r, pl.ds(cc * 16, 16)] = jnp.maximum(z, 0.1 * z)
            return rcarry

        lax.fori_loop(0, CH, row_body, 0)

        pltpu.make_async_copy(dstg.at[pl.ds(base0, CH)],
                              sdbufs[b], sdsems[b]).wait()
        pltpu.async_copy(gath_v, agg.at[sdbufs[b]], sctsems[b], add=True)

    def chunk_body(i, carry):
        process(3 * i, 0)
        process(3 * i + 1, 1)
        process(3 * i + 2, 2)
        return carry

    lax.fori_loop(0, (NCHUNK - 1) // 3, chunk_body, 0)
    process(NCHUNK - 1, (NCHUNK - 1) % 3)
    wait_sct((NCHUNK - 3) % 3)
    wait_sct((NCHUNK - 2) % 3)
    wait_sct((NCHUNK - 1) % 3)
    plsc.subcore_barrier()

    def wb_body(j, carry):
        k = s + NS * j

        @pl.when(k < NG)
        def _():
            pltpu.sync_copy(agg.at[pl.ds(k * G, G)],
                            out.at[c, pl.ds(k * G, G)])

        return carry

    lax.fori_loop(0, (NG + NS - 1) // NS, wb_body, 0)


_sc_aggregate = pl.kernel(
    _sc_body,
    out_type=jax.ShapeDtypeStruct((NC, N, D), jnp.float32),
    mesh=plsc.VectorSubcoreMesh(core_axis_name="c", subcore_axis_name="s",
                                num_cores=NC, num_subcores=NS),
    scratch_types=(
        [pltpu.VMEM((CH,), jnp.int32)] * 6
        + [pltpu.VMEM((CH, D), jnp.float32)] * 6
        + [pltpu.VMEM_SHARED((N, D), jnp.float32)]
        + [pltpu.SemaphoreType.DMA] * 15
    ),
)


def _prep_body(x_r, wp_r, bp_r, u2a_r, h_r, ha_r):
    h = jnp.dot(x_r[...], wp_r[...], preferred_element_type=jnp.float32)
    h = h + bp_r[...]
    h_r[...] = h
    ha_r[...] = jnp.dot(h, u2a_r[...], preferred_element_type=jnp.float32)


_tc_prep = pl.pallas_call(
    _prep_body,
    grid=(N // NB,),
    in_specs=[
        pl.BlockSpec((NB, D), lambda i: (i, 0)),
        pl.BlockSpec((D, D), lambda i: (0, 0)),
        pl.BlockSpec((1, D), lambda i: (0, 0)),
        pl.BlockSpec((D, D), lambda i: (0, 0)),
    ],
    out_specs=[
        pl.BlockSpec((NB, D), lambda i: (i, 0)),
        pl.BlockSpec((NB, D), lambda i: (i, 0)),
    ],
    out_shape=[
        jax.ShapeDtypeStruct((N, D), jnp.float32),
        jax.ShapeDtypeStruct((N, D), jnp.float32),
    ],
)


def _eb_body(ea_r, u2e_r, b2_r, eb_r):
    eb_r[...] = (jnp.dot(ea_r[...], u2e_r[...],
                         preferred_element_type=jnp.float32) + b2_r[...])


_tc_eb = pl.pallas_call(
    _eb_body,
    grid=(E // BE,),
    in_specs=[
        pl.BlockSpec((BE, DE), lambda j: (j, 0)),
        pl.BlockSpec((DE, D), lambda j: (0, 0)),
        pl.BlockSpec((1, D), lambda j: (0, 0)),
    ],
    out_specs=pl.BlockSpec((BE, D), lambda j: (j, 0)),
    out_shape=jax.ShapeDtypeStruct((E, D), jnp.float32),
)


def _upd_body_mid(h_r, agg_r, u1a_r, u1b_r, b1_r, u2a_r, hn_r, han_r):
    a = agg_r[0] + agg_r[1]
    hn = jnp.dot(h_r[...], u1a_r[...], preferred_element_type=jnp.float32)
    hn = hn + jnp.dot(a, u1b_r[...], preferred_element_type=jnp.float32)
    hn = hn + b1_r[...]
    hn_r[...] = hn
    han_r[...] = jnp.dot(hn, u2a_r[...], preferred_element_type=jnp.float32)


_tc_update_mid = pl.pallas_call(
    _upd_body_mid,
    grid=(N // NB,),
    in_specs=[
        pl.BlockSpec((NB, D), lambda i: (i, 0)),
        pl.BlockSpec((NC, NB, D), lambda i: (0, i, 0)),
        pl.BlockSpec((D, D), lambda i: (0, 0)),
        pl.BlockSpec((D, D), lambda i: (0, 0)),
        pl.BlockSpec((1, D), lambda i: (0, 0)),
        pl.BlockSpec((D, D), lambda i: (0, 0)),
    ],
    out_specs=[
        pl.BlockSpec((NB, D), lambda i: (i, 0)),
        pl.BlockSpec((NB, D), lambda i: (i, 0)),
    ],
    out_shape=[
        jax.ShapeDtypeStruct((N, D), jnp.float32),
        jax.ShapeDtypeStruct((N, D), jnp.float32),
    ],
)


def _upd_body_last(h_r, agg_r, u1a_r, u1b_r, b1_r, hn_r):
    a = agg_r[0] + agg_r[1]
    hn = jnp.dot(h_r[...], u1a_r[...], preferred_element_type=jnp.float32)
    hn = hn + jnp.dot(a, u1b_r[...], preferred_element_type=jnp.float32)
    hn_r[...] = hn + b1_r[...]


_tc_update_last = pl.pallas_call(
    _upd_body_last,
    grid=(N // NB,),
    in_specs=[
        pl.BlockSpec((NB, D), lambda i: (i, 0)),
        pl.BlockSpec((NC, NB, D), lambda i: (0, i, 0)),
        pl.BlockSpec((D, D), lambda i: (0, 0)),
        pl.BlockSpec((D, D), lambda i: (0, 0)),
        pl.BlockSpec((1, D), lambda i: (0, 0)),
    ],
    out_specs=pl.BlockSpec((NB, D), lambda i: (i, 0)),
    out_shape=jax.ShapeDtypeStruct((N, D), jnp.float32),
)


def kernel(x, edge_index, edge_attr, W_proj, b_proj, U2_W, U2_b, U1_W, U1_b):
    src = edge_index[0]
    dst = edge_index[1]
    bp = b_proj.reshape(1, D)
    zeros = jnp.zeros((N, D), jnp.float32)

    h, hA = _tc_prep(x, W_proj, bp, U2_W[0, :D, :])
    eBs = [_tc_eb(edge_attr, U2_W[i, D:, :], U2_b[i].reshape(1, D))
           for i in range(DEPTH)]
    for i in range(DEPTH):
        agg = _sc_aggregate(hA, src, dst, eBs[i], zeros)
        if i < DEPTH - 1:
            h, hA = _tc_update_mid(h, agg, U1_W[i, :D, :], U1_W[i, D:, :],
                                   U1_b[i].reshape(1, D), U2_W[i + 1, :D, :])
        else:
            h = _tc_update_last(h, agg, U1_W[i, :D, :], U1_W[i, D:, :],
                                U1_b[i].reshape(1, D))
    return h
